# baseline (speedup 1.0000x reference)
import jax
import jax.numpy as jnp
import numpy as np
from jax.experimental import pallas as pl
from jax.experimental.pallas import tpu as pltpu

_D = 128
_DB = 32

_G = [1.0, float(np.exp(-0.5)), float(np.exp(-2.0))]


def _m_matrix():
    def g(e):
        e = abs(e)
        return _G[e] if e <= 2 else 0.0

    m = np.zeros((_D, _D), np.float32)
    for d in range(_D):
        for q in range(_D):
            k = sum(g(d + dd - q) for dd in (-1, 0, 1) if 0 <= d + dd < _D)
            m[d, q] = k / 27.0
    return m


_M = _m_matrix()
_M_HI = _M.astype(jnp.bfloat16)
_M_LO = (_M - _M_HI.astype(np.float32)).astype(jnp.bfloat16)


def _splat_pool_kernel(depth_ref, xray_ref, mhi_ref, mlo_ref, out_ref, x_ref):
    db = pl.program_id(1)
    HW = depth_ref.shape[-1]

    @pl.when(db == 0)
    def _():
        depth = depth_ref[0]
        di = jnp.clip((depth * (_D - 1)).astype(jnp.int32), 0, _D - 1)
        q = jax.lax.broadcasted_iota(jnp.int32, (_D, HW), 0)
        x_ref[...] = (q == di).astype(jnp.bfloat16)

    x = x_ref[...]
    w = jax.lax.dot_general(
        mhi_ref[...], x,
        dimension_numbers=(((1,), (0,)), ((), ())),
        preferred_element_type=jnp.float32)
    w = w + jax.lax.dot_general(
        mlo_ref[...], x,
        dimension_numbers=(((1,), (0,)), ((), ())),
        preferred_element_type=jnp.float32)

    p = w * xray_ref[0]

    zh = jnp.zeros((_DB, 128), jnp.float32)
    t = p + jnp.concatenate([zh, p[:, :-128]], axis=1) \
          + jnp.concatenate([p[:, 128:], zh], axis=1)

    lane = jax.lax.broadcasted_iota(jnp.int32, (1, HW), 1) % 128
    zw = jnp.zeros((_DB, 1), jnp.float32)
    sl = jnp.concatenate([zw, t[:, :-1]], axis=1)
    sr = jnp.concatenate([t[:, 1:], zw], axis=1)
    s = t + jnp.where(lane != 0, sl, 0.0) + jnp.where(lane != 127, sr, 0.0)
    out_ref[0] = s


def kernel(depth_map, x_ray):
    B, _, H, W = depth_map.shape
    HW = H * W
    out = pl.pallas_call(
        _splat_pool_kernel,
        grid=(B, _D // _DB),
        in_specs=[
            pl.BlockSpec((1, 1, HW), lambda b, d: (b, 0, 0)),
            pl.BlockSpec((1, 1, HW), lambda b, d: (b, 0, 0)),
            pl.BlockSpec((_DB, _D), lambda b, d: (d, 0)),
            pl.BlockSpec((_DB, _D), lambda b, d: (d, 0)),
        ],
        out_specs=pl.BlockSpec((1, _DB, HW), lambda b, d: (b, d, 0)),
        out_shape=jax.ShapeDtypeStruct((B, _D, HW), jnp.float32),
        scratch_shapes=[pltpu.VMEM((_D, HW), jnp.bfloat16)],
    )(depth_map.reshape(B, 1, HW), x_ray.reshape(B, 1, HW),
      jnp.asarray(_M_HI), jnp.asarray(_M_LO))
    return out.reshape(B, 1, _D, H, W)


# speedup vs baseline: 2.2313x; 2.2313x over previous
import jax
import jax.numpy as jnp
import numpy as np
from jax.experimental import pallas as pl

_D = 128
_DB = 128

_G1 = float(np.exp(-0.5))
_G2 = float(np.exp(-2.0))


def _box_hw(x, H, W):
    n = x.shape[0]
    zh = jnp.zeros((n, 1, W), jnp.float32)
    u = x + jnp.concatenate([zh, x[:, :-1, :]], axis=1) \
          + jnp.concatenate([x[:, 1:, :], zh], axis=1)
    zw = jnp.zeros((n, H, 1), jnp.float32)
    return u + jnp.concatenate([zw, u[:, :, :-1]], axis=2) \
             + jnp.concatenate([u[:, :, 1:], zw], axis=2)


def _splat_pool_kernel(depth_ref, xray_ref, out_ref):
    db = pl.program_id(1)
    nd = pl.num_programs(1)
    depth = depth_ref[0, 0]
    inten = xray_ref[0, 0]
    H, W = depth.shape
    di = jnp.clip((depth * (_D - 1)).astype(jnp.int32), 0, _D - 1)
    i27 = inten * (1.0 / 27.0)

    q = (db * _DB - 3) + jax.lax.broadcasted_iota(jnp.int32, (_DB + 6, 1, 1), 0)
    m = jnp.where(q == di[None, :, :], i27[None, :, :], 0.0)

    g = m[2:-2] + _G1 * (m[1:-3] + m[3:-1]) + _G2 * (m[:-4] + m[4:])

    t = g[1:-1] + g[:-2] + g[2:]
    out_ref[0] = _box_hw(t, H, W)

    @pl.when(db == 0)
    def _():
        out_ref[0, 0:1] -= _box_hw(g[0:1], H, W)

    @pl.when(db == nd - 1)
    def _():
        out_ref[0, _DB - 1:_DB] -= _box_hw(g[_DB + 1:_DB + 2], H, W)


def kernel(depth_map, x_ray):
    B, _, H, W = depth_map.shape
    out = pl.pallas_call(
        _splat_pool_kernel,
        grid=(B, _D // _DB),
        in_specs=[
            pl.BlockSpec((1, 1, H, W), lambda b, d: (b, 0, 0, 0)),
            pl.BlockSpec((1, 1, H, W), lambda b, d: (b, 0, 0, 0)),
        ],
        out_specs=pl.BlockSpec((1, _DB, H, W), lambda b, d: (b, d, 0, 0)),
        out_shape=jax.ShapeDtypeStruct((B, _D, H, W), jnp.float32),
    )(depth_map, x_ray)
    return out[:, None]


# speedup vs baseline: 2.2408x; 1.0043x over previous
import jax
import jax.numpy as jnp
import numpy as np
from jax.experimental import pallas as pl

_D = 128

_G1 = float(np.exp(-0.5))
_G2 = float(np.exp(-2.0))


def _splat_pool_kernel(depth_ref, xray_ref, out_ref):
    depth = depth_ref[0, 0]
    inten = xray_ref[0, 0]
    H, W = depth.shape
    di = jnp.clip((depth * (_D - 1)).astype(jnp.int32), 0, _D - 1)
    i27 = inten * (1.0 / 27.0)

    q = jax.lax.broadcasted_iota(jnp.int32, (_D, 1, 1), 0)
    m = jnp.where(q == di[None, :, :], i27[None, :, :], 0.0)

    z1 = jnp.zeros((1, H, W), jnp.float32)
    z2 = jnp.zeros((2, H, W), jnp.float32)
    mm1 = jnp.concatenate([z1, m[:-1]], axis=0)
    mp1 = jnp.concatenate([m[1:], z1], axis=0)
    mm2 = jnp.concatenate([z2, m[:-2]], axis=0)
    mp2 = jnp.concatenate([m[2:], z2], axis=0)
    g = m + _G1 * (mm1 + mp1) + _G2 * (mm2 + mp2)

    t = g + jnp.concatenate([z1, g[:-1]], axis=0) \
          + jnp.concatenate([g[1:], z1], axis=0)

    zh = jnp.zeros((_D, 1, W), jnp.float32)
    u = t + jnp.concatenate([zh, t[:, :-1, :]], axis=1) \
          + jnp.concatenate([t[:, 1:, :], zh], axis=1)

    zw = jnp.zeros((_D, H, 1), jnp.float32)
    out_ref[0] = u + jnp.concatenate([zw, u[:, :, :-1]], axis=2) \
                   + jnp.concatenate([u[:, :, 1:], zw], axis=2)


def kernel(depth_map, x_ray):
    B, _, H, W = depth_map.shape
    out = pl.pallas_call(
        _splat_pool_kernel,
        grid=(B,),
        in_specs=[
            pl.BlockSpec((1, 1, H, W), lambda b: (b, 0, 0, 0)),
            pl.BlockSpec((1, 1, H, W), lambda b: (b, 0, 0, 0)),
        ],
        out_specs=pl.BlockSpec((1, _D, H, W), lambda b: (b, 0, 0, 0)),
        out_shape=jax.ShapeDtypeStruct((B, _D, H, W), jnp.float32),
    )(depth_map, x_ray)
    return out[:, None]
